# baseline (speedup 1.0000x reference)
import functools

import jax
import jax.numpy as jnp
from jax import lax
from jax.experimental import pallas as pl
from jax.experimental.pallas import tpu as pltpu
from jax.experimental.pallas import tpu_sc as plsc


def _routing_sc(router_logits):
    T, E = router_logits.shape
    info = plsc.get_sparse_core_info()
    NW = info.num_cores * info.num_subcores
    rows_per = T // NW

    mesh = plsc.VectorSubcoreMesh(core_axis_name="c", subcore_axis_name="s")

    @functools.partial(
        pl.kernel,
        mesh=mesh,
        out_type=jax.ShapeDtypeStruct((T, E), jnp.float32),
        scratch_types=[
            pltpu.VMEM((rows_per, E), jnp.float32),
            pltpu.VMEM((rows_per, E), jnp.float32),
        ],
        compiler_params=pltpu.CompilerParams(needs_layout_passes=False),
    )
    def k(logits_hbm, out_hbm, in_v, out_v):
        wid = lax.axis_index("s") * info.num_cores + lax.axis_index("c")
        base = wid * rows_per
        pltpu.sync_copy(logits_hbm.at[pl.ds(base, rows_per)], in_v)
        iot = lax.iota(jnp.int32, 16)
        for t in range(rows_per):
            v = in_v[t]
            m = jnp.max(v)
            ex = jnp.exp(v - m)
            m1 = jnp.max(ex)
            i1 = jnp.min(jnp.where(ex == m1, iot, E))
            ex2 = jnp.where(iot == i1, jnp.float32(-1.0), ex)
            m2 = jnp.max(ex2)
            i2 = jnp.min(jnp.where(ex2 == m2, iot, E))
            num = jnp.where(
                iot == i1, m1, jnp.where(iot == i2, m2, jnp.float32(0.0))
            )
            den = jnp.broadcast_to(m1 + m2, (16,))
            out_v[t] = num / den
        pltpu.sync_copy(out_v, out_hbm.at[pl.ds(base, rows_per)])

    return k(router_logits)


def _moe_tc(x, full_w, gate_w, up_w, down_w):
    T, H = x.shape
    E, I, _ = gate_w.shape
    TI = 128
    NI = I // TI

    def body(x_ref, w_ref, g_ref, u_ref, d_ref, o_ref):
        e = pl.program_id(0)
        i = pl.program_id(1)

        @pl.when((e == 0) & (i == 0))
        def _init():
            o_ref[...] = jnp.zeros_like(o_ref)

        xv = x_ref[...]
        g = lax.dot_general(
            xv, g_ref[0], (((1,), (1,)), ((), ())),
            preferred_element_type=jnp.float32,
        )
        u = lax.dot_general(
            xv, u_ref[0], (((1,), (1,)), ((), ())),
            preferred_element_type=jnp.float32,
        )
        h = g * jax.nn.sigmoid(g) * u
        po = lax.dot_general(
            h, d_ref[0], (((1,), (1,)), ((), ())),
            preferred_element_type=jnp.float32,
        )
        lane = lax.broadcasted_iota(jnp.int32, (T, E), 1)
        wcol = jnp.sum(
            jnp.where(lane == e, w_ref[...], jnp.float32(0.0)),
            axis=1,
            keepdims=True,
        )
        o_ref[...] += wcol * po

    return pl.pallas_call(
        body,
        grid=(E, NI),
        in_specs=[
            pl.BlockSpec((T, H), lambda e, i: (0, 0)),
            pl.BlockSpec((T, E), lambda e, i: (0, 0)),
            pl.BlockSpec((1, TI, H), lambda e, i: (e, i, 0)),
            pl.BlockSpec((1, TI, H), lambda e, i: (e, i, 0)),
            pl.BlockSpec((1, H, TI), lambda e, i: (e, 0, i)),
        ],
        out_specs=pl.BlockSpec((T, H), lambda e, i: (0, 0)),
        out_shape=jax.ShapeDtypeStruct((T, H), jnp.float32),
        compiler_params=pltpu.CompilerParams(
            dimension_semantics=("arbitrary", "arbitrary"),
        ),
    )(x, full_w, gate_w, up_w, down_w)


@jax.jit
def kernel(x, router_logits, gate_w, up_w, down_w):
    full_w = _routing_sc(router_logits)
    return _moe_tc(x, full_w, gate_w, up_w, down_w)


# speedup vs baseline: 1.0117x; 1.0117x over previous
import functools

import jax
import jax.numpy as jnp
from jax import lax
from jax.experimental import pallas as pl
from jax.experimental.pallas import tpu as pltpu
from jax.experimental.pallas import tpu_sc as plsc


def _routing_sc(router_logits):
    T, E = router_logits.shape
    info = plsc.get_sparse_core_info()
    NW = info.num_cores * info.num_subcores
    rows_per = T // NW

    mesh = plsc.VectorSubcoreMesh(core_axis_name="c", subcore_axis_name="s")

    @functools.partial(
        pl.kernel,
        mesh=mesh,
        out_type=jax.ShapeDtypeStruct((T, E), jnp.float32),
        scratch_types=[
            pltpu.VMEM((rows_per, E), jnp.float32),
            pltpu.VMEM((rows_per, E), jnp.float32),
        ],
        compiler_params=pltpu.CompilerParams(needs_layout_passes=False),
    )
    def k(logits_hbm, out_hbm, in_v, out_v):
        wid = lax.axis_index("s") * info.num_cores + lax.axis_index("c")
        base = wid * rows_per
        pltpu.sync_copy(logits_hbm.at[pl.ds(base, rows_per)], in_v)
        iot = lax.iota(jnp.int32, 16)
        for t in range(rows_per):
            v = in_v[t]
            m = jnp.max(v)
            ex = jnp.exp(v - m)
            m1 = jnp.max(ex)
            i1 = jnp.min(jnp.where(ex == m1, iot, E))
            ex2 = jnp.where(iot == i1, jnp.float32(-1.0), ex)
            m2 = jnp.max(ex2)
            i2 = jnp.min(jnp.where(ex2 == m2, iot, E))
            num = jnp.where(
                iot == i1, m1, jnp.where(iot == i2, m2, jnp.float32(0.0))
            )
            den = jnp.broadcast_to(m1 + m2, (16,))
            out_v[t] = num / den
        pltpu.sync_copy(out_v, out_hbm.at[pl.ds(base, rows_per)])

    return k(router_logits)


def _moe_tc(x, full_w, gate_w, up_w, down_w):
    T, H = x.shape
    E, I, _ = gate_w.shape
    TI = 128
    NI = I // TI

    def body(x_ref, w_ref, g_ref, u_ref, d_ref, o_ref, h_ref):
        e = pl.program_id(0)
        i = pl.program_id(1)

        @pl.when((e == 0) & (i == 0))
        def _init():
            o_ref[...] = jnp.zeros_like(o_ref)

        xv = x_ref[...]
        g = lax.dot_general(
            xv, g_ref[0], (((1,), (1,)), ((), ())),
            preferred_element_type=jnp.float32,
        )
        u = lax.dot_general(
            xv, u_ref[0], (((1,), (1,)), ((), ())),
            preferred_element_type=jnp.float32,
        )
        h_ref[i] = g * jax.nn.sigmoid(g) * u

        @pl.when(i == NI - 1)
        def _down():
            dt = d_ref[0]
            po = lax.dot_general(
                h_ref[0], dt[:, 0:TI], (((1,), (1,)), ((), ())),
                preferred_element_type=jnp.float32,
            )
            for j in range(1, NI):
                po += lax.dot_general(
                    h_ref[j], dt[:, j * TI:(j + 1) * TI],
                    (((1,), (1,)), ((), ())),
                    preferred_element_type=jnp.float32,
                )
            lane = lax.broadcasted_iota(jnp.int32, (T, E), 1)
            wcol = jnp.sum(
                jnp.where(lane == e, w_ref[...], jnp.float32(0.0)),
                axis=1,
                keepdims=True,
            )
            o_ref[...] += wcol * po

    return pl.pallas_call(
        body,
        grid=(E, NI),
        in_specs=[
            pl.BlockSpec((T, H), lambda e, i: (0, 0)),
            pl.BlockSpec((T, E), lambda e, i: (0, 0)),
            pl.BlockSpec((1, TI, H), lambda e, i: (e, i, 0)),
            pl.BlockSpec((1, TI, H), lambda e, i: (e, i, 0)),
            pl.BlockSpec((1, H, I), lambda e, i: (e, 0, 0)),
        ],
        out_specs=pl.BlockSpec((T, H), lambda e, i: (0, 0)),
        out_shape=jax.ShapeDtypeStruct((T, H), jnp.float32),
        scratch_shapes=[pltpu.VMEM((NI, T, TI), jnp.float32)],
        compiler_params=pltpu.CompilerParams(
            dimension_semantics=("arbitrary", "arbitrary"),
        ),
    )(x, full_w, gate_w, up_w, down_w)


@jax.jit
def kernel(x, router_logits, gate_w, up_w, down_w):
    full_w = _routing_sc(router_logits)
    return _moe_tc(x, full_w, gate_w, up_w, down_w)
